# baseline (speedup 1.0000x reference)
import dataclasses
import functools

import jax
import jax.numpy as jnp
from jax import lax
from jax.experimental import pallas as pl
from jax.experimental.pallas import tpu as pltpu
from jax.experimental.pallas import tpu_sc as plsc

_CLS_W = 1.0
_BBOX_W = 10.0
_BS = 8
_A = 9
_HW = 2500
_N = _A * _HW

_NC = 2
_NS = 16
_NW = _NC * _NS
_T = 256
_TPI = 10
_NT = _BS * _TPI
_TAIL_BB0 = 2304
_TAIL_GT0 = 2248
_TAIL_GOFF = _TAIL_BB0 - _TAIL_GT0
_TAIL_PHI = _HW - _TAIL_BB0


def _sl1_acc(acc, bvec, gvec, mvec):
    d = bvec - gvec
    ad = jnp.abs(d)
    sl1 = jnp.where(ad < 1.0, 0.5 * d * d, ad - 0.5)
    return acc + sl1 * mvec


def _sc_tile(bb_v, gt_v, lb_v, acc_ref, goff, phi, n_vec):
    lanes = lax.iota(jnp.int32, 16)
    zero = jnp.zeros((16,), jnp.float32)

    @pl.loop(0, n_vec)
    def _(v):
        q = lanes + v * 16
        valid = q < phi
        gidx = jnp.minimum(q + goff, _T - 1)
        masks = []
        for a in range(_A):
            la = plsc.load_gather(lb_v, [gidx, jnp.full((16,), a, jnp.int32)])
            masks.append(la.astype(jnp.float32))
        acc = acc_ref[...]
        for ch in range(4 * _A):
            bvec = bb_v[ch, pl.ds(v * 16, 16)]
            gvec = plsc.load_gather(
                gt_v, [gidx, jnp.full((16,), ch, jnp.int32)])
            d = bvec - gvec
            ad = jnp.abs(d)
            sl1 = jnp.where(ad < 1.0, 0.5 * d * d, ad - 0.5)
            acc = acc + jnp.where(valid, sl1 * masks[ch // 4], zero)
        acc_ref[...] = acc


def _sc_body(bbox_hbm, gt_hbm, labels_hbm, out_hbm,
             bb_v, gt_v, lb_v, acc_v, sem):
    cid = lax.axis_index("c")
    sid = lax.axis_index("s")
    wid = sid * _NC + cid

    acc_v[...] = jnp.zeros((16,), jnp.float32)

    @pl.loop(0, (_NT + _NW - 1) // _NW)
    def _(k):
        t = wid + k * _NW

        @pl.when(t < _NT)
        def _():
            b = t // _TPI
            j = t % _TPI
            is_tail = j == _TPI - 1
            bb_p0 = jnp.where(is_tail, _TAIL_BB0, j * _T)
            gt_p0 = jnp.where(is_tail, _TAIL_GT0, j * _T)
            goff = jnp.where(is_tail, _TAIL_GOFF, 0)
            phi = jnp.where(is_tail, _TAIL_PHI, _T)
            n_vec = jnp.where(is_tail, (_TAIL_PHI + 15) // 16, _T // 16)
            bb_p0 = pl.multiple_of(bb_p0, 128)
            gt_p0 = pl.multiple_of(gt_p0, 8)

            pltpu.async_copy(bbox_hbm.at[b, :, pl.ds(bb_p0, _T)],
                             bb_v, sem).wait()
            pltpu.async_copy(gt_hbm.at[b, pl.ds(gt_p0, _T)], gt_v,
                             sem).wait()
            pltpu.async_copy(labels_hbm.at[b, pl.ds(gt_p0, _T)], lb_v,
                             sem).wait()
            _sc_tile(bb_v, gt_v, lb_v, acc_v, goff, phi, n_vec)

    pltpu.sync_copy(acc_v, out_hbm.at[cid, sid])


def _sc_compiler_params():
    cp = pltpu.CompilerParams()
    if "needs_layout_passes" in pltpu.CompilerParams.__dataclass_fields__:
        cp = dataclasses.replace(cp, needs_layout_passes=False)
    return cp


@functools.partial(jax.jit, static_argnames=())
def _sc_masked_sl1(bbox, gt, labels_p):
    kern = pl.kernel(
        _sc_body,
        out_type=jax.ShapeDtypeStruct((_NC, _NS, 16), jnp.float32),
        mesh=plsc.VectorSubcoreMesh(core_axis_name="c", subcore_axis_name="s"),
        compiler_params=_sc_compiler_params(),
        scratch_types=[
            pltpu.VMEM((4 * _A, _T), jnp.float32),
            pltpu.VMEM((_T, 4 * _A), jnp.float32),
            pltpu.VMEM((_T, _A), jnp.int32),
            pltpu.VMEM((16,), jnp.float32),
            pltpu.SemaphoreType.DMA,
        ],
    )
    return kern(bbox, gt, labels_p)


def _bce_body(logits_ref, labels_ref, out_ref):
    lg = logits_ref[...]
    tg = labels_ref[...].astype(jnp.float32)
    out_ref[0, 0] = jnp.sum(
        jnp.maximum(lg, 0.0) - lg * tg + jnp.log1p(jnp.exp(-jnp.abs(lg))))
    out_ref[0, 1] = jnp.sum(tg)


def kernel(rpn_cls_logits, rpn_bbox_reg, anchor_labels, anchor_gt_boxes):
    logits = rpn_cls_logits.reshape(_BS * _A, _HW)
    labels_a = anchor_labels.reshape(_BS * _A, _HW)
    labels_p = anchor_labels.reshape(_BS, _HW, _A)
    bbox = rpn_bbox_reg.reshape(_BS, 4 * _A, _HW)
    gt = anchor_gt_boxes.reshape(_BS, _HW, 4 * _A)

    partials = _sc_masked_sl1(bbox, gt, labels_p)

    bce = pl.pallas_call(
        _bce_body,
        out_shape=jax.ShapeDtypeStruct((1, 2), jnp.float32),
        out_specs=pl.BlockSpec(memory_space=pltpu.SMEM),
    )(logits, labels_a)

    bce_sum = bce[0, 0]
    npos = bce[0, 1]
    masked_sum = jnp.sum(partials)
    cls_loss = bce_sum / (_BS * _N)
    denom = jnp.maximum(2.0 * npos, 1.0)
    bbox_loss = jnp.where(npos > 0.0, masked_sum / denom, 0.0)
    return _CLS_W * cls_loss + _BBOX_W * bbox_loss


# speedup vs baseline: 1.0732x; 1.0732x over previous
import dataclasses
import functools

import jax
import jax.numpy as jnp
from jax import lax
from jax.experimental import pallas as pl
from jax.experimental.pallas import tpu as pltpu
from jax.experimental.pallas import tpu_sc as plsc

_CLS_W = 1.0
_BBOX_W = 10.0
_BS = 8
_A = 9
_HW = 2500
_N = _A * _HW

_NC = 2
_NS = 16
_NW = _NC * _NS
_T = 128
_TPI = 20
_NT = _BS * _TPI
_TPW = _NT // _NW
_TAIL_BB0 = 2432
_TAIL_GT0 = 2376
_TAIL_GOFF = _TAIL_BB0 - _TAIL_GT0
_TAIL_PHI = _HW - _TAIL_BB0


def _sl1_acc(acc, bvec, gvec, mvec):
    d = bvec - gvec
    ad = jnp.abs(d)
    sl1 = jnp.where(ad < 1.0, 0.5 * d * d, ad - 0.5)
    return acc + sl1 * mvec


def _sc_tile(bb_v, gt_v, lb_v, acc_ref, goff, phi, n_vec):
    lanes = lax.iota(jnp.int32, 16)
    zero = jnp.zeros((16,), jnp.float32)

    @pl.loop(0, n_vec)
    def _(v):
        q = lanes + v * 16
        valid = q < phi
        gidx = jnp.minimum(q + goff, _T - 1)
        masks = []
        for a in range(_A):
            la = plsc.load_gather(lb_v, [gidx, jnp.full((16,), a, jnp.int32)])
            masks.append(la.astype(jnp.float32))
        acc = acc_ref[...]
        for ch in range(4 * _A):
            bvec = bb_v[ch, pl.ds(v * 16, 16)]
            gvec = plsc.load_gather(
                gt_v, [gidx, jnp.full((16,), ch, jnp.int32)])
            d = bvec - gvec
            ad = jnp.abs(d)
            sl1 = jnp.where(ad < 1.0, 0.5 * d * d, ad - 0.5)
            acc = acc + jnp.where(valid, sl1 * masks[ch // 4], zero)
        acc_ref[...] = acc


def _tile_params(t):
    b = t // _TPI
    j = t % _TPI
    is_tail = j == _TPI - 1
    bb_p0 = pl.multiple_of(jnp.where(is_tail, _TAIL_BB0, j * _T), 128)
    gt_p0 = pl.multiple_of(jnp.where(is_tail, _TAIL_GT0, j * _T), 8)
    goff = jnp.where(is_tail, _TAIL_GOFF, 0)
    phi = jnp.where(is_tail, _TAIL_PHI, _T)
    n_vec = jnp.where(is_tail, (_TAIL_PHI + 15) // 16, _T // 16)
    return b, bb_p0, gt_p0, goff, phi, n_vec


def _sc_body(bbox_hbm, gt_hbm, labels_hbm, out_hbm,
             bb_v, gt_v, lb_v, acc_v, sems):
    cid = lax.axis_index("c")
    sid = lax.axis_index("s")
    wid = sid * _NC + cid

    acc_v[...] = jnp.zeros((16,), jnp.float32)

    def start(k, buf):
        b, bb_p0, gt_p0, _, _, _ = _tile_params(wid + k * _NW)
        copies = (
            pltpu.make_async_copy(bbox_hbm.at[b, :, pl.ds(bb_p0, _T)],
                                  bb_v.at[buf], sems.at[buf, 0]),
            pltpu.make_async_copy(gt_hbm.at[b, pl.ds(gt_p0, _T)],
                                  gt_v.at[buf], sems.at[buf, 1]),
            pltpu.make_async_copy(labels_hbm.at[b, pl.ds(gt_p0, _T)],
                                  lb_v.at[buf], sems.at[buf, 2]),
        )
        for c in copies:
            c.start()
        return copies

    inflight = start(0, 0)
    for k in range(_TPW):
        for c in inflight:
            c.wait()
        buf = k % 2
        if k + 1 < _TPW:
            inflight = start(k + 1, (k + 1) % 2)
        _, _, _, goff, phi, n_vec = _tile_params(wid + k * _NW)
        _sc_tile(bb_v.at[buf], gt_v.at[buf], lb_v.at[buf], acc_v,
                 goff, phi, n_vec)

    pltpu.sync_copy(acc_v, out_hbm.at[cid, sid])


def _sc_compiler_params():
    cp = pltpu.CompilerParams()
    if "needs_layout_passes" in pltpu.CompilerParams.__dataclass_fields__:
        cp = dataclasses.replace(cp, needs_layout_passes=False)
    return cp


@functools.partial(jax.jit, static_argnames=())
def _sc_masked_sl1(bbox, gt, labels_p):
    kern = pl.kernel(
        _sc_body,
        out_type=jax.ShapeDtypeStruct((_NC, _NS, 16), jnp.float32),
        mesh=plsc.VectorSubcoreMesh(core_axis_name="c", subcore_axis_name="s"),
        compiler_params=_sc_compiler_params(),
        scratch_types=[
            pltpu.VMEM((2, 4 * _A, _T), jnp.float32),
            pltpu.VMEM((2, _T, 4 * _A), jnp.float32),
            pltpu.VMEM((2, _T, _A), jnp.int32),
            pltpu.VMEM((16,), jnp.float32),
            pltpu.SemaphoreType.DMA((2, 3)),
        ],
    )
    return kern(bbox, gt, labels_p)


def _bce_body(logits_ref, labels_ref, out_ref):
    lg = logits_ref[...]
    tg = labels_ref[...].astype(jnp.float32)
    out_ref[0, 0] = jnp.sum(
        jnp.maximum(lg, 0.0) - lg * tg + jnp.log1p(jnp.exp(-jnp.abs(lg))))
    out_ref[0, 1] = jnp.sum(tg)


def kernel(rpn_cls_logits, rpn_bbox_reg, anchor_labels, anchor_gt_boxes):
    logits = rpn_cls_logits.reshape(_BS * _A, _HW)
    labels_a = anchor_labels.reshape(_BS * _A, _HW)
    labels_p = anchor_labels.reshape(_BS, _HW, _A)
    bbox = rpn_bbox_reg.reshape(_BS, 4 * _A, _HW)
    gt = anchor_gt_boxes.reshape(_BS, _HW, 4 * _A)

    partials = _sc_masked_sl1(bbox, gt, labels_p)

    bce = pl.pallas_call(
        _bce_body,
        out_shape=jax.ShapeDtypeStruct((1, 2), jnp.float32),
        out_specs=pl.BlockSpec(memory_space=pltpu.SMEM),
    )(logits, labels_a)

    bce_sum = bce[0, 0]
    npos = bce[0, 1]
    masked_sum = jnp.sum(partials)
    cls_loss = bce_sum / (_BS * _N)
    denom = jnp.maximum(2.0 * npos, 1.0)
    bbox_loss = jnp.where(npos > 0.0, masked_sum / denom, 0.0)
    return _CLS_W * cls_loss + _BBOX_W * bbox_loss


# speedup vs baseline: 1.1196x; 1.0432x over previous
import dataclasses
import functools

import jax
import jax.numpy as jnp
from jax import lax
from jax.experimental import pallas as pl
from jax.experimental.pallas import tpu as pltpu
from jax.experimental.pallas import tpu_sc as plsc

_CLS_W = 1.0
_BBOX_W = 10.0
_BS = 8
_A = 9
_HW = 2500
_N = _A * _HW

_NC = 2
_NS = 16
_NW = _NC * _NS
_T = 128
_TPI = 20
_SCB = 2
_NT = _SCB * _TPI
_TAIL_BB0 = 2432
_TAIL_GT0 = 2376
_TAIL_GOFF = _TAIL_BB0 - _TAIL_GT0
_TAIL_PHI = _HW - _TAIL_BB0


def _sl1_acc(acc, bvec, gvec, mvec):
    d = bvec - gvec
    ad = jnp.abs(d)
    sl1 = jnp.where(ad < 1.0, 0.5 * d * d, ad - 0.5)
    return acc + sl1 * mvec


def _sc_tile(bb_v, gt_v, lb_v, acc_ref, goff, phi, n_vec):
    lanes = lax.iota(jnp.int32, 16)
    zero = jnp.zeros((16,), jnp.float32)

    @pl.loop(0, n_vec)
    def _(v):
        q = lanes + v * 16
        valid = q < phi
        gidx = jnp.minimum(q + goff, _T - 1)
        masks = []
        for a in range(_A):
            la = plsc.load_gather(lb_v, [gidx, jnp.full((16,), a, jnp.int32)])
            masks.append(la.astype(jnp.float32))
        acc = acc_ref[...]
        for ch in range(4 * _A):
            bvec = bb_v[ch, pl.ds(v * 16, 16)]
            gvec = plsc.load_gather(
                gt_v, [gidx, jnp.full((16,), ch, jnp.int32)])
            d = bvec - gvec
            ad = jnp.abs(d)
            sl1 = jnp.where(ad < 1.0, 0.5 * d * d, ad - 0.5)
            acc = acc + jnp.where(valid, sl1 * masks[ch // 4], zero)
        acc_ref[...] = acc


def _tile_params(t):
    b = t // _TPI
    j = t % _TPI
    is_tail = j == _TPI - 1
    bb_p0 = pl.multiple_of(jnp.where(is_tail, _TAIL_BB0, j * _T), 128)
    gt_p0 = pl.multiple_of(jnp.where(is_tail, _TAIL_GT0, j * _T), 8)
    goff = jnp.where(is_tail, _TAIL_GOFF, 0)
    phi = jnp.where(is_tail, _TAIL_PHI, _T)
    n_vec = jnp.where(is_tail, (_TAIL_PHI + 15) // 16, _T // 16)
    return b, bb_p0, gt_p0, goff, phi, n_vec


def _sc_body(bbox_hbm, gt_hbm, labels_hbm, out_hbm,
             bb_v, gt_v, lb_v, acc_v, sems):
    cid = lax.axis_index("c")
    sid = lax.axis_index("s")
    wid = sid * _NC + cid

    acc_v[...] = jnp.zeros((16,), jnp.float32)

    def process(k, buf):
        t = wid + k * _NW
        b, bb_p0, gt_p0, goff, phi, n_vec = _tile_params(t)
        copies = (
            pltpu.make_async_copy(bbox_hbm.at[b, :, pl.ds(bb_p0, _T)],
                                  bb_v.at[buf], sems.at[buf, 0]),
            pltpu.make_async_copy(gt_hbm.at[b, pl.ds(gt_p0, _T)],
                                  gt_v.at[buf], sems.at[buf, 1]),
            pltpu.make_async_copy(labels_hbm.at[b, pl.ds(gt_p0, _T)],
                                  lb_v.at[buf], sems.at[buf, 2]),
        )
        for c in copies:
            c.start()
        for c in copies:
            c.wait()
        _sc_tile(bb_v.at[buf], gt_v.at[buf], lb_v.at[buf], acc_v,
                 goff, phi, n_vec)

    process(0, 0)
    for k in range(1, (_NT + _NW - 1) // _NW):

        @pl.when(wid + k * _NW < _NT)
        def _(k=k):
            process(k, k % 2)

    pltpu.sync_copy(acc_v, out_hbm.at[cid, sid])


def _sc_compiler_params():
    cp = pltpu.CompilerParams()
    if "needs_layout_passes" in pltpu.CompilerParams.__dataclass_fields__:
        cp = dataclasses.replace(cp, needs_layout_passes=False)
    return cp


@functools.partial(jax.jit, static_argnames=())
def _sc_masked_sl1(bbox, gt, labels_p):
    kern = pl.kernel(
        _sc_body,
        out_type=jax.ShapeDtypeStruct((_NC, _NS, 16), jnp.float32),
        mesh=plsc.VectorSubcoreMesh(core_axis_name="c", subcore_axis_name="s"),
        compiler_params=_sc_compiler_params(),
        scratch_types=[
            pltpu.VMEM((2, 4 * _A, _T), jnp.float32),
            pltpu.VMEM((2, _T, 4 * _A), jnp.float32),
            pltpu.VMEM((2, _T, _A), jnp.int32),
            pltpu.VMEM((16,), jnp.float32),
            pltpu.SemaphoreType.DMA((2, 3)),
        ],
    )
    return kern(bbox, gt, labels_p)


def _tc_sl1_body(bbox_ref, gt_ref, labels_ref, out_ref, acc_ref):
    i = pl.program_id(0)

    @pl.when(i == 0)
    def _():
        acc_ref[0] = jnp.float32(0.0)

    mp = labels_ref[0].astype(jnp.float32)
    a_i = jax.lax.broadcasted_iota(jnp.int32, (_A, 4 * _A), 0)
    ch_i = jax.lax.broadcasted_iota(jnp.int32, (_A, 4 * _A), 1)
    pmat = (a_i == ch_i // 4).astype(jnp.float32)
    mask36 = jax.lax.dot_general(
        mp, pmat, dimension_numbers=(((1,), (0,)), ((), ())),
        preferred_element_type=jnp.float32)

    bt = jnp.transpose(bbox_ref[0], (1, 0))
    diff = bt - gt_ref[0]
    ad = jnp.abs(diff)
    sl1 = jnp.where(ad < 1.0, 0.5 * diff * diff, ad - 0.5)
    acc_ref[0] += jnp.sum(sl1 * mask36)

    @pl.when(i == pl.num_programs(0) - 1)
    def _():
        out_ref[0, 0] = acc_ref[0]


def _bce_body(logits_ref, labels_ref, out_ref):
    lg = logits_ref[...]
    tg = labels_ref[...].astype(jnp.float32)
    out_ref[0, 0] = jnp.sum(
        jnp.maximum(lg, 0.0) - lg * tg + jnp.log1p(jnp.exp(-jnp.abs(lg))))
    out_ref[0, 1] = jnp.sum(tg)


def kernel(rpn_cls_logits, rpn_bbox_reg, anchor_labels, anchor_gt_boxes):
    logits = rpn_cls_logits.reshape(_BS * _A, _HW)
    labels_a = anchor_labels.reshape(_BS * _A, _HW)
    labels_p = anchor_labels.reshape(_BS, _HW, _A)
    bbox = rpn_bbox_reg.reshape(_BS, 4 * _A, _HW)
    gt = anchor_gt_boxes.reshape(_BS, _HW, 4 * _A)

    partials = _sc_masked_sl1(bbox, gt, labels_p)

    n_tc = _BS - _SCB
    tc_sl1 = pl.pallas_call(
        _tc_sl1_body,
        grid=(n_tc,),
        in_specs=[
            pl.BlockSpec((1, 4 * _A, _HW), lambda i: (i + _SCB, 0, 0)),
            pl.BlockSpec((1, _HW, 4 * _A), lambda i: (i + _SCB, 0, 0)),
            pl.BlockSpec((1, _HW, _A), lambda i: (i + _SCB, 0, 0)),
        ],
        out_shape=jax.ShapeDtypeStruct((1, 1), jnp.float32),
        out_specs=pl.BlockSpec(memory_space=pltpu.SMEM),
        scratch_shapes=[pltpu.SMEM((1,), jnp.float32)],
    )(bbox, gt, labels_p)

    bce = pl.pallas_call(
        _bce_body,
        out_shape=jax.ShapeDtypeStruct((1, 2), jnp.float32),
        out_specs=pl.BlockSpec(memory_space=pltpu.SMEM),
    )(logits, labels_a)

    bce_sum = bce[0, 0]
    npos = bce[0, 1]
    masked_sum = jnp.sum(partials) + tc_sl1[0, 0]
    cls_loss = bce_sum / (_BS * _N)
    denom = jnp.maximum(2.0 * npos, 1.0)
    bbox_loss = jnp.where(npos > 0.0, masked_sum / denom, 0.0)
    return _CLS_W * cls_loss + _BBOX_W * bbox_loss


# speedup vs baseline: 1.2439x; 1.1110x over previous
import dataclasses
import functools

import jax
import jax.numpy as jnp
from jax import lax
from jax.experimental import pallas as pl
from jax.experimental.pallas import tpu as pltpu
from jax.experimental.pallas import tpu_sc as plsc

_CLS_W = 1.0
_BBOX_W = 10.0
_BS = 8
_A = 9
_HW = 2500
_N = _A * _HW

_NC = 2
_NS = 16
_NW = _NC * _NS
_T = 128
_TPI = 20
_SCB = 0
_NT = _SCB * _TPI
_TAIL_BB0 = 2432
_TAIL_GT0 = 2376
_TAIL_GOFF = _TAIL_BB0 - _TAIL_GT0
_TAIL_PHI = _HW - _TAIL_BB0


def _sl1_acc(acc, bvec, gvec, mvec):
    d = bvec - gvec
    ad = jnp.abs(d)
    sl1 = jnp.where(ad < 1.0, 0.5 * d * d, ad - 0.5)
    return acc + sl1 * mvec


def _sc_tile(bb_v, gt_v, lb_v, acc_ref, goff, phi, n_vec):
    lanes = lax.iota(jnp.int32, 16)
    zero = jnp.zeros((16,), jnp.float32)

    @pl.loop(0, n_vec)
    def _(v):
        q = lanes + v * 16
        valid = q < phi
        gidx = jnp.minimum(q + goff, _T - 1)
        masks = []
        for a in range(_A):
            la = plsc.load_gather(lb_v, [gidx, jnp.full((16,), a, jnp.int32)])
            masks.append(la.astype(jnp.float32))
        acc = acc_ref[...]
        for ch in range(4 * _A):
            bvec = bb_v[ch, pl.ds(v * 16, 16)]
            gvec = plsc.load_gather(
                gt_v, [gidx, jnp.full((16,), ch, jnp.int32)])
            d = bvec - gvec
            ad = jnp.abs(d)
            sl1 = jnp.where(ad < 1.0, 0.5 * d * d, ad - 0.5)
            acc = acc + jnp.where(valid, sl1 * masks[ch // 4], zero)
        acc_ref[...] = acc


def _tile_params(t):
    b = t // _TPI
    j = t % _TPI
    is_tail = j == _TPI - 1
    bb_p0 = pl.multiple_of(jnp.where(is_tail, _TAIL_BB0, j * _T), 128)
    gt_p0 = pl.multiple_of(jnp.where(is_tail, _TAIL_GT0, j * _T), 8)
    goff = jnp.where(is_tail, _TAIL_GOFF, 0)
    phi = jnp.where(is_tail, _TAIL_PHI, _T)
    n_vec = jnp.where(is_tail, (_TAIL_PHI + 15) // 16, _T // 16)
    return b, bb_p0, gt_p0, goff, phi, n_vec


def _sc_body(bbox_hbm, gt_hbm, labels_hbm, out_hbm,
             bb_v, gt_v, lb_v, acc_v, sems):
    cid = lax.axis_index("c")
    sid = lax.axis_index("s")
    wid = sid * _NC + cid

    acc_v[...] = jnp.zeros((16,), jnp.float32)

    def process(k, buf):
        t = wid + k * _NW
        b, bb_p0, gt_p0, goff, phi, n_vec = _tile_params(t)
        copies = (
            pltpu.make_async_copy(bbox_hbm.at[b, :, pl.ds(bb_p0, _T)],
                                  bb_v.at[buf], sems.at[buf, 0]),
            pltpu.make_async_copy(gt_hbm.at[b, pl.ds(gt_p0, _T)],
                                  gt_v.at[buf], sems.at[buf, 1]),
            pltpu.make_async_copy(labels_hbm.at[b, pl.ds(gt_p0, _T)],
                                  lb_v.at[buf], sems.at[buf, 2]),
        )
        for c in copies:
            c.start()
        for c in copies:
            c.wait()
        _sc_tile(bb_v.at[buf], gt_v.at[buf], lb_v.at[buf], acc_v,
                 goff, phi, n_vec)

    process(0, 0)
    for k in range(1, (_NT + _NW - 1) // _NW):

        @pl.when(wid + k * _NW < _NT)
        def _(k=k):
            process(k, k % 2)

    pltpu.sync_copy(acc_v, out_hbm.at[cid, sid])


def _sc_compiler_params():
    cp = pltpu.CompilerParams()
    if "needs_layout_passes" in pltpu.CompilerParams.__dataclass_fields__:
        cp = dataclasses.replace(cp, needs_layout_passes=False)
    return cp


@functools.partial(jax.jit, static_argnames=())
def _sc_masked_sl1(bbox, gt, labels_p):
    kern = pl.kernel(
        _sc_body,
        out_type=jax.ShapeDtypeStruct((_NC, _NS, 16), jnp.float32),
        mesh=plsc.VectorSubcoreMesh(core_axis_name="c", subcore_axis_name="s"),
        compiler_params=_sc_compiler_params(),
        scratch_types=[
            pltpu.VMEM((2, 4 * _A, _T), jnp.float32),
            pltpu.VMEM((2, _T, 4 * _A), jnp.float32),
            pltpu.VMEM((2, _T, _A), jnp.int32),
            pltpu.VMEM((16,), jnp.float32),
            pltpu.SemaphoreType.DMA((2, 3)),
        ],
    )
    return kern(bbox, gt, labels_p)


def _tc_sl1_body(bbox_ref, gt_ref, labels_ref, out_ref, acc_ref):
    i = pl.program_id(0)

    @pl.when(i == 0)
    def _():
        acc_ref[0] = jnp.float32(0.0)

    mp = labels_ref[0].astype(jnp.float32)
    a_i = jax.lax.broadcasted_iota(jnp.int32, (_A, 4 * _A), 0)
    ch_i = jax.lax.broadcasted_iota(jnp.int32, (_A, 4 * _A), 1)
    pmat = (a_i == ch_i // 4).astype(jnp.float32)
    mask36 = jax.lax.dot_general(
        mp, pmat, dimension_numbers=(((1,), (0,)), ((), ())),
        preferred_element_type=jnp.float32)

    bt = jnp.transpose(bbox_ref[0], (1, 0))
    diff = bt - gt_ref[0]
    ad = jnp.abs(diff)
    sl1 = jnp.where(ad < 1.0, 0.5 * diff * diff, ad - 0.5)
    acc_ref[0] += jnp.sum(sl1 * mask36)

    @pl.when(i == pl.num_programs(0) - 1)
    def _():
        out_ref[0, 0] = acc_ref[0]


def _bce_body(logits_ref, labels_ref, out_ref):
    lg = logits_ref[...]
    tg = labels_ref[...].astype(jnp.float32)
    out_ref[0, 0] = jnp.sum(
        jnp.maximum(lg, 0.0) - lg * tg + jnp.log1p(jnp.exp(-jnp.abs(lg))))
    out_ref[0, 1] = jnp.sum(tg)


def kernel(rpn_cls_logits, rpn_bbox_reg, anchor_labels, anchor_gt_boxes):
    logits = rpn_cls_logits.reshape(_BS * _A, _HW)
    labels_a = anchor_labels.reshape(_BS * _A, _HW)
    labels_p = anchor_labels.reshape(_BS, _HW, _A)
    bbox = rpn_bbox_reg.reshape(_BS, 4 * _A, _HW)
    gt = anchor_gt_boxes.reshape(_BS, _HW, 4 * _A)

    partials = (_sc_masked_sl1(bbox, gt, labels_p) if _SCB
                else jnp.zeros((_NC, _NS, 16), jnp.float32))

    n_tc = _BS - _SCB
    tc_sl1 = pl.pallas_call(
        _tc_sl1_body,
        grid=(n_tc,),
        in_specs=[
            pl.BlockSpec((1, 4 * _A, _HW), lambda i: (i + _SCB, 0, 0)),
            pl.BlockSpec((1, _HW, 4 * _A), lambda i: (i + _SCB, 0, 0)),
            pl.BlockSpec((1, _HW, _A), lambda i: (i + _SCB, 0, 0)),
        ],
        out_shape=jax.ShapeDtypeStruct((1, 1), jnp.float32),
        out_specs=pl.BlockSpec(memory_space=pltpu.SMEM),
        scratch_shapes=[pltpu.SMEM((1,), jnp.float32)],
    )(bbox, gt, labels_p)

    bce = pl.pallas_call(
        _bce_body,
        out_shape=jax.ShapeDtypeStruct((1, 2), jnp.float32),
        out_specs=pl.BlockSpec(memory_space=pltpu.SMEM),
    )(logits, labels_a)

    bce_sum = bce[0, 0]
    npos = bce[0, 1]
    masked_sum = jnp.sum(partials) + tc_sl1[0, 0]
    cls_loss = bce_sum / (_BS * _N)
    denom = jnp.maximum(2.0 * npos, 1.0)
    bbox_loss = jnp.where(npos > 0.0, masked_sum / denom, 0.0)
    return _CLS_W * cls_loss + _BBOX_W * bbox_loss


# speedup vs baseline: 1.2855x; 1.0334x over previous
import jax
import jax.numpy as jnp
from jax.experimental import pallas as pl
from jax.experimental.pallas import tpu as pltpu

_CLS_W = 1.0
_BBOX_W = 10.0
_BS = 8
_A = 9
_HW = 2500
_N = _A * _HW


def _loss_body(logits_ref, labels_a_ref, labels_p_ref, bbox_ref, gt_ref,
               out_ref, acc_ref):
    i = pl.program_id(0)

    @pl.when(i == 0)
    def _():
        acc_ref[0] = jnp.float32(0.0)
        acc_ref[1] = jnp.float32(0.0)
        acc_ref[2] = jnp.float32(0.0)

    lg = logits_ref[0]
    tg = labels_a_ref[0].astype(jnp.float32)
    acc_ref[0] += jnp.sum(
        jnp.maximum(lg, 0.0) - lg * tg + jnp.log1p(jnp.exp(-jnp.abs(lg))))
    acc_ref[1] += jnp.sum(tg)

    mp = labels_p_ref[0].astype(jnp.float32)
    a_i = jax.lax.broadcasted_iota(jnp.int32, (_A, 4 * _A), 0)
    ch_i = jax.lax.broadcasted_iota(jnp.int32, (_A, 4 * _A), 1)
    pmat = (a_i == ch_i // 4).astype(jnp.float32)
    mask36 = jax.lax.dot_general(
        mp, pmat, dimension_numbers=(((1,), (0,)), ((), ())),
        preferred_element_type=jnp.float32)

    bt = jnp.transpose(bbox_ref[0], (1, 0))
    diff = bt - gt_ref[0]
    ad = jnp.abs(diff)
    sl1 = jnp.where(ad < 1.0, 0.5 * diff * diff, ad - 0.5)
    acc_ref[2] += jnp.sum(sl1 * mask36)

    @pl.when(i == _BS - 1)
    def _():
        cls_loss = acc_ref[0] / (_BS * _N)
        npos = acc_ref[1]
        denom = jnp.maximum(2.0 * npos, 1.0)
        bbox_loss = jnp.where(npos > 0.0, acc_ref[2] / denom, 0.0)
        out_ref[0, 0] = _CLS_W * cls_loss + _BBOX_W * bbox_loss


def kernel(rpn_cls_logits, rpn_bbox_reg, anchor_labels, anchor_gt_boxes):
    logits = rpn_cls_logits.reshape(_BS, _A, _HW)
    labels_a = anchor_labels.reshape(_BS, _A, _HW)
    labels_p = anchor_labels.reshape(_BS, _HW, _A)
    bbox = rpn_bbox_reg.reshape(_BS, 4 * _A, _HW)
    gt = anchor_gt_boxes.reshape(_BS, _HW, 4 * _A)

    out = pl.pallas_call(
        _loss_body,
        grid=(_BS,),
        in_specs=[
            pl.BlockSpec((1, _A, _HW), lambda i: (i, 0, 0)),
            pl.BlockSpec((1, _A, _HW), lambda i: (i, 0, 0)),
            pl.BlockSpec((1, _HW, _A), lambda i: (i, 0, 0)),
            pl.BlockSpec((1, 4 * _A, _HW), lambda i: (i, 0, 0)),
            pl.BlockSpec((1, _HW, 4 * _A), lambda i: (i, 0, 0)),
        ],
        out_shape=jax.ShapeDtypeStruct((1, 1), jnp.float32),
        out_specs=pl.BlockSpec(memory_space=pltpu.SMEM),
        scratch_shapes=[pltpu.SMEM((3,), jnp.float32)],
    )(logits, labels_a, labels_p, bbox, gt)
    return out[0, 0]
